# Initial kernel scaffold; baseline (speedup 1.0000x reference)
import functools

import jax
import jax.numpy as jnp
from jax import lax
from jax.experimental import pallas as pl
from jax.experimental.pallas import tpu as pltpu
from jax.experimental.pallas import tpu_sc as plsc

N = 10000
E = 320000
D_IN = 128
D_HID = 128
D_OUT = 64

NUM_CORES = 2
NUM_SUBCORES = 16
NUM_TILES = NUM_CORES * NUM_SUBCORES
EPT = E // NUM_TILES
K = 80
NCH = EPT // K
RPT = N // NUM_SUBCORES
ZR = 125

_MESH = plsc.VectorSubcoreMesh(core_axis_name="c", subcore_axis_name="s")


def _zero_acc(zbuf, acc, sid, d):
    def _zrow(i, carry):
        for j in range(d // 16):
            zbuf[i, pl.ds(j * 16, 16)] = jnp.zeros((16,), jnp.float32)
        return carry

    lax.fori_loop(0, ZR, _zrow, 0)
    for z in range(RPT // ZR):
        pltpu.sync_copy(zbuf, acc.at[pl.ds(sid * RPT + z * ZR, ZR)])


def _deg_body(col_hbm, ew_hbm, out_hbm, cidx, ewv, rows, zbuf, acc, sem):
    cid = lax.axis_index("c")
    sid = lax.axis_index("s")
    wid = cid * NUM_SUBCORES + sid

    _zero_acc(zbuf, acc, sid, 16)
    plsc.subcore_barrier()

    pltpu.sync_copy(col_hbm.at[wid], cidx)
    pltpu.sync_copy(ew_hbm.at[wid], ewv)

    def chunk(i, carry):
        def fill(e, c2):
            rows[e, :] = jnp.full((16,), ewv[i, e], jnp.float32)
            return c2

        lax.fori_loop(0, K, fill, 0)
        pltpu.sync_copy(rows, acc.at[cidx.at[i]], add=True)
        return carry

    lax.fori_loop(0, NCH, chunk, 0)
    plsc.subcore_barrier()
    pltpu.sync_copy(acc.at[pl.ds(sid * RPT, RPT)],
                    out_hbm.at[cid, pl.ds(sid * RPT, RPT)])


_deg_kernel = functools.partial(
    pl.kernel,
    out_type=jax.ShapeDtypeStruct((NUM_CORES, N, 16), jnp.float32),
    mesh=_MESH,
    scratch_types=[
        pltpu.VMEM((NCH, K), jnp.int32),
        pltpu.VMEM((NCH, K), jnp.float32),
        pltpu.VMEM((K, 16), jnp.float32),
        pltpu.VMEM((ZR, 16), jnp.float32),
        pltpu.VMEM_SHARED((N, 16), jnp.float32),
        pltpu.SemaphoreType.DMA,
    ],
)(_deg_body)


def _msg_body(d, y_hbm, row_hbm, col_hbm, ew_hbm, out_hbm,
              ridx, cidx, ewv, rows, zbuf, acc, sem):
    cid = lax.axis_index("c")
    sid = lax.axis_index("s")
    wid = cid * NUM_SUBCORES + sid

    _zero_acc(zbuf, acc, sid, d)
    plsc.subcore_barrier()

    pltpu.sync_copy(row_hbm.at[wid], ridx)
    pltpu.sync_copy(col_hbm.at[wid], cidx)
    pltpu.sync_copy(ew_hbm.at[wid], ewv)

    def chunk(i, carry):
        pltpu.async_copy(y_hbm.at[ridx.at[i]], rows, sem).wait()

        def scale(e, c2):
            w = ewv[i, e]
            for j in range(d // 16):
                rows[e, pl.ds(j * 16, 16)] = rows[e, pl.ds(j * 16, 16)] * w
            return c2

        lax.fori_loop(0, K, scale, 0)
        pltpu.sync_copy(rows, acc.at[cidx.at[i]], add=True)
        return carry

    lax.fori_loop(0, NCH, chunk, 0)
    plsc.subcore_barrier()
    pltpu.sync_copy(acc.at[pl.ds(sid * RPT, RPT)],
                    out_hbm.at[cid, pl.ds(sid * RPT, RPT)])


def _make_msg_kernel(d):
    return functools.partial(
        pl.kernel,
        out_type=jax.ShapeDtypeStruct((NUM_CORES, N, d), jnp.float32),
        mesh=_MESH,
        scratch_types=[
            pltpu.VMEM((NCH, K), jnp.int32),
            pltpu.VMEM((NCH, K), jnp.int32),
            pltpu.VMEM((NCH, K), jnp.float32),
            pltpu.VMEM((K, d), jnp.float32),
            pltpu.VMEM((ZR, d), jnp.float32),
            pltpu.VMEM_SHARED((N, d), jnp.float32),
            pltpu.SemaphoreType.DMA,
        ],
    )(functools.partial(_msg_body, d))


_msg_kernel_128 = _make_msg_kernel(D_HID)
_msg_kernel_64 = _make_msg_kernel(D_OUT)


_R = 1000
_G = N // _R


def _mm_body(x_ref, w_ref, o_ref):
    o_ref[...] = jnp.dot(x_ref[...], w_ref[...],
                         preferred_element_type=jnp.float32)


def _matmul(x, w):
    return pl.pallas_call(
        _mm_body,
        grid=(_G,),
        in_specs=[
            pl.BlockSpec((_R, x.shape[1]), lambda i: (i, 0)),
            pl.BlockSpec(w.shape, lambda i: (0, 0)),
        ],
        out_specs=pl.BlockSpec((_R, w.shape[1]), lambda i: (i, 0)),
        out_shape=jax.ShapeDtypeStruct((x.shape[0], w.shape[1]),
                                       jnp.float32),
    )(x, w)


def _dinv_of(degp_ref):
    deg = degp_ref[0, :, :1] + degp_ref[1, :, :1] + 1.0
    return lax.rsqrt(deg)


def _prescale_body(xw_ref, degp_ref, y_ref):
    y_ref[...] = xw_ref[...] * _dinv_of(degp_ref)


def _prescale(xw, degp):
    d = xw.shape[1]
    return pl.pallas_call(
        _prescale_body,
        grid=(_G,),
        in_specs=[
            pl.BlockSpec((_R, d), lambda i: (i, 0)),
            pl.BlockSpec((NUM_CORES, _R, 16), lambda i: (0, i, 0)),
        ],
        out_specs=pl.BlockSpec((_R, d), lambda i: (i, 0)),
        out_shape=jax.ShapeDtypeStruct((N, d), jnp.float32),
    )(xw, degp)


def _layer_body(aggp_ref, xw_ref, degp_ref, b_ref, w_ref, xw2_ref, y2_ref):
    dinv = _dinv_of(degp_ref)
    s = (dinv * (aggp_ref[0] + aggp_ref[1])
         + (dinv * dinv) * xw_ref[...] + b_ref[...])
    h = jnp.maximum(s, 0.0)
    xw2 = jnp.dot(h, w_ref[...], preferred_element_type=jnp.float32)
    xw2_ref[...] = xw2
    y2_ref[...] = xw2 * dinv


def _layer(aggp, xw1, degp, b1, w2):
    d_in = xw1.shape[1]
    d_out = w2.shape[1]
    return pl.pallas_call(
        _layer_body,
        grid=(_G,),
        in_specs=[
            pl.BlockSpec((NUM_CORES, _R, d_in), lambda i: (0, i, 0)),
            pl.BlockSpec((_R, d_in), lambda i: (i, 0)),
            pl.BlockSpec((NUM_CORES, _R, 16), lambda i: (0, i, 0)),
            pl.BlockSpec((1, d_in), lambda i: (0, 0)),
            pl.BlockSpec((d_in, d_out), lambda i: (0, 0)),
        ],
        out_specs=[
            pl.BlockSpec((_R, d_out), lambda i: (i, 0)),
            pl.BlockSpec((_R, d_out), lambda i: (i, 0)),
        ],
        out_shape=[
            jax.ShapeDtypeStruct((N, d_out), jnp.float32),
            jax.ShapeDtypeStruct((N, d_out), jnp.float32),
        ],
    )(aggp, xw1, degp, b1, w2)


def _final_body(aggp_ref, xw_ref, degp_ref, b_ref, o_ref):
    dinv = _dinv_of(degp_ref)
    o_ref[...] = (dinv * (aggp_ref[0] + aggp_ref[1])
                  + (dinv * dinv) * xw_ref[...] + b_ref[...])


def _final(aggp, xw2, degp, b2):
    d = xw2.shape[1]
    return pl.pallas_call(
        _final_body,
        grid=(_G,),
        in_specs=[
            pl.BlockSpec((NUM_CORES, _R, d), lambda i: (0, i, 0)),
            pl.BlockSpec((_R, d), lambda i: (i, 0)),
            pl.BlockSpec((NUM_CORES, _R, 16), lambda i: (0, i, 0)),
            pl.BlockSpec((1, d), lambda i: (0, 0)),
        ],
        out_specs=pl.BlockSpec((_R, d), lambda i: (i, 0)),
        out_shape=jax.ShapeDtypeStruct((N, d), jnp.float32),
    )(aggp, xw2, degp, b2)


def kernel(x, edge_index, edge_attr, W1, b1, W2, b2):
    row = edge_index[0].reshape(NUM_TILES, NCH, K)
    col = edge_index[1].reshape(NUM_TILES, NCH, K)
    ew = edge_attr.reshape(NUM_TILES, NCH, K)
    b1r = b1.reshape(1, D_HID)
    b2r = b2.reshape(1, D_OUT)

    degp = _deg_kernel(col, ew)
    xw1 = _matmul(x, W1)
    y1 = _prescale(xw1, degp)
    aggp1 = _msg_kernel_128(y1, row, col, ew)
    xw2, y2 = _layer(aggp1, xw1, degp, b1r, W2)
    aggp2 = _msg_kernel_64(y2, row, col, ew)
    out = _final(aggp2, xw2, degp, b2r)
    return out


# speedup vs baseline: 9.3876x; 9.3876x over previous
import functools

import jax
import jax.numpy as jnp
from jax import lax
from jax.experimental import pallas as pl
from jax.experimental.pallas import tpu as pltpu
from jax.experimental.pallas import tpu_sc as plsc

N = 10000
E = 320000
D_IN = 128
D_HID = 128
D_OUT = 64

NUM_CORES = 2
NUM_SUBCORES = 16
NUM_TILES = NUM_CORES * NUM_SUBCORES
EPT = E // NUM_TILES
K = 80
NCH = EPT // K
RPT = N // NUM_SUBCORES
ZR = 125
WB = 624
WB_TAIL = N - NUM_SUBCORES * WB

_MESH = plsc.VectorSubcoreMesh(core_axis_name="c", subcore_axis_name="s")
_SC_PARAMS = pltpu.CompilerParams(use_tc_tiling_on_sc=False)


def _zero_acc(zbuf, acc, sid, d):
    def _zrow(i, carry):
        for j in range(d // 16):
            zbuf[i, pl.ds(j * 16, 16)] = jnp.zeros((16,), jnp.float32)
        return carry

    lax.fori_loop(0, ZR, _zrow, 0)
    for z in range(RPT // ZR):
        pltpu.sync_copy(zbuf, acc.at[pl.ds(sid * RPT + z * ZR, ZR)])


def _deg_body(col_hbm, ew_hbm, out_hbm, cidx, ewv, rows, zbuf, acc, sem):
    cid = lax.axis_index("c")
    sid = lax.axis_index("s")
    wid = cid * NUM_SUBCORES + sid

    _zero_acc(zbuf, acc, sid, 16)
    plsc.subcore_barrier()

    pltpu.sync_copy(col_hbm.at[wid], cidx)
    pltpu.sync_copy(ew_hbm.at[wid], ewv)

    def chunk(i, carry):
        def fill(g, c2):
            wv = ewv[i, pl.ds(g * 16, 16)]
            for j in range(16):
                rows[g * 16 + j, :] = jnp.full((16,), wv[j], jnp.float32)
            return c2

        lax.fori_loop(0, K // 16, fill, 0)
        pltpu.sync_copy(rows, acc.at[cidx.at[i]], add=True)
        return carry

    lax.fori_loop(0, NCH, chunk, 0)
    plsc.subcore_barrier()
    pltpu.sync_copy(acc.at[pl.ds(sid * WB, WB)],
                    out_hbm.at[cid, pl.ds(sid * WB, WB)])

    @pl.when(sid == 0)
    def _tail():
        pltpu.sync_copy(acc.at[pl.ds(NUM_SUBCORES * WB, WB_TAIL)],
                        out_hbm.at[cid, pl.ds(NUM_SUBCORES * WB, WB_TAIL)])


_deg_kernel = functools.partial(
    pl.kernel,
    out_type=jax.ShapeDtypeStruct((NUM_CORES, N, 16), jnp.float32),
    mesh=_MESH,
    scratch_types=[
        pltpu.VMEM((NCH, K), jnp.int32),
        pltpu.VMEM((NCH, K), jnp.float32),
        pltpu.VMEM((K, 16), jnp.float32),
        pltpu.VMEM((ZR, 16), jnp.float32),
        pltpu.VMEM_SHARED((N, 16), jnp.float32),
        pltpu.SemaphoreType.DMA,
    ],
    compiler_params=_SC_PARAMS,
)(_deg_body)


def _msg_body(d, y_hbm, row_hbm, col_hbm, ew_hbm, out_hbm,
              ridx, cidx, ewv, rows, zbuf, acc, sem):
    cid = lax.axis_index("c")
    sid = lax.axis_index("s")
    wid = cid * NUM_SUBCORES + sid

    _zero_acc(zbuf, acc, sid, d)
    plsc.subcore_barrier()

    pltpu.sync_copy(row_hbm.at[wid], ridx)
    pltpu.sync_copy(col_hbm.at[wid], cidx)
    pltpu.sync_copy(ew_hbm.at[wid], ewv)

    def chunk(i, carry):
        pltpu.async_copy(y_hbm.at[ridx.at[i]], rows, sem).wait()

        def scale(g, c2):
            wv = ewv[i, pl.ds(g * 16, 16)]
            for u in range(16):
                e = g * 16 + u
                w = wv[u]
                for j in range(d // 16):
                    rows[e, pl.ds(j * 16, 16)] = (
                        rows[e, pl.ds(j * 16, 16)] * w)
            return c2

        lax.fori_loop(0, K // 16, scale, 0)
        pltpu.sync_copy(rows, acc.at[cidx.at[i]], add=True)
        return carry

    lax.fori_loop(0, NCH, chunk, 0)
    plsc.subcore_barrier()
    pltpu.sync_copy(acc.at[pl.ds(sid * WB, WB)],
                    out_hbm.at[cid, pl.ds(sid * WB, WB)])

    @pl.when(sid == 0)
    def _tail():
        pltpu.sync_copy(acc.at[pl.ds(NUM_SUBCORES * WB, WB_TAIL)],
                        out_hbm.at[cid, pl.ds(NUM_SUBCORES * WB, WB_TAIL)])


def _make_msg_kernel(d):
    return functools.partial(
        pl.kernel,
        out_type=jax.ShapeDtypeStruct((NUM_CORES, N, d), jnp.float32),
        mesh=_MESH,
        scratch_types=[
            pltpu.VMEM((NCH, K), jnp.int32),
            pltpu.VMEM((NCH, K), jnp.int32),
            pltpu.VMEM((NCH, K), jnp.float32),
            pltpu.VMEM((K, d), jnp.float32),
            pltpu.VMEM((ZR, d), jnp.float32),
            pltpu.VMEM_SHARED((N, d), jnp.float32),
            pltpu.SemaphoreType.DMA,
        ],
        compiler_params=_SC_PARAMS,
    )(functools.partial(_msg_body, d))


_msg_kernel_64 = _make_msg_kernel(D_OUT)


_R = 1000
_G = N // _R


def _mm_body(x_ref, w_ref, o_ref):
    o_ref[...] = jnp.dot(x_ref[...], w_ref[...],
                         preferred_element_type=jnp.float32)


def _matmul(x, w):
    return pl.pallas_call(
        _mm_body,
        grid=(_G,),
        in_specs=[
            pl.BlockSpec((_R, x.shape[1]), lambda i: (i, 0)),
            pl.BlockSpec(w.shape, lambda i: (0, 0)),
        ],
        out_specs=pl.BlockSpec((_R, w.shape[1]), lambda i: (i, 0)),
        out_shape=jax.ShapeDtypeStruct((x.shape[0], w.shape[1]),
                                       jnp.float32),
    )(x, w)


def _dinv_of(degp_ref):
    deg = degp_ref[0, :, :1] + degp_ref[1, :, :1] + 1.0
    return lax.rsqrt(deg)


def _prescale_body(xw_ref, degp_ref, ya_ref, yb_ref):
    y = xw_ref[...] * _dinv_of(degp_ref)
    ya_ref[...] = y[:, :D_OUT]
    yb_ref[...] = y[:, D_OUT:]


def _prescale(xw, degp):
    d = xw.shape[1]
    return pl.pallas_call(
        _prescale_body,
        grid=(_G,),
        in_specs=[
            pl.BlockSpec((_R, d), lambda i: (i, 0)),
            pl.BlockSpec((NUM_CORES, _R, 16), lambda i: (0, i, 0)),
        ],
        out_specs=[
            pl.BlockSpec((_R, D_OUT), lambda i: (i, 0)),
            pl.BlockSpec((_R, D_OUT), lambda i: (i, 0)),
        ],
        out_shape=[
            jax.ShapeDtypeStruct((N, D_OUT), jnp.float32),
            jax.ShapeDtypeStruct((N, D_OUT), jnp.float32),
        ],
    )(xw, degp)


def _layer_body(aggpa_ref, aggpb_ref, xw_ref, degp_ref, b_ref, w_ref,
                xw2_ref, y2_ref):
    dinv = _dinv_of(degp_ref)
    agg = jnp.concatenate(
        [aggpa_ref[0] + aggpa_ref[1], aggpb_ref[0] + aggpb_ref[1]], axis=1)
    s = dinv * agg + (dinv * dinv) * xw_ref[...] + b_ref[...]
    h = jnp.maximum(s, 0.0)
    xw2 = jnp.dot(h, w_ref[...], preferred_element_type=jnp.float32)
    xw2_ref[...] = xw2
    y2_ref[...] = xw2 * dinv


def _layer(aggpa, aggpb, xw1, degp, b1, w2):
    d_in = xw1.shape[1]
    d_out = w2.shape[1]
    return pl.pallas_call(
        _layer_body,
        grid=(_G,),
        in_specs=[
            pl.BlockSpec((NUM_CORES, _R, D_OUT), lambda i: (0, i, 0)),
            pl.BlockSpec((NUM_CORES, _R, D_OUT), lambda i: (0, i, 0)),
            pl.BlockSpec((_R, d_in), lambda i: (i, 0)),
            pl.BlockSpec((NUM_CORES, _R, 16), lambda i: (0, i, 0)),
            pl.BlockSpec((1, d_in), lambda i: (0, 0)),
            pl.BlockSpec((d_in, d_out), lambda i: (0, 0)),
        ],
        out_specs=[
            pl.BlockSpec((_R, d_out), lambda i: (i, 0)),
            pl.BlockSpec((_R, d_out), lambda i: (i, 0)),
        ],
        out_shape=[
            jax.ShapeDtypeStruct((N, d_out), jnp.float32),
            jax.ShapeDtypeStruct((N, d_out), jnp.float32),
        ],
    )(aggpa, aggpb, xw1, degp, b1, w2)


def _final_body(aggp_ref, xw_ref, degp_ref, b_ref, o_ref):
    dinv = _dinv_of(degp_ref)
    o_ref[...] = (dinv * (aggp_ref[0] + aggp_ref[1])
                  + (dinv * dinv) * xw_ref[...] + b_ref[...])


def _final(aggp, xw2, degp, b2):
    d = xw2.shape[1]
    return pl.pallas_call(
        _final_body,
        grid=(_G,),
        in_specs=[
            pl.BlockSpec((NUM_CORES, _R, d), lambda i: (0, i, 0)),
            pl.BlockSpec((_R, d), lambda i: (i, 0)),
            pl.BlockSpec((NUM_CORES, _R, 16), lambda i: (0, i, 0)),
            pl.BlockSpec((1, d), lambda i: (0, 0)),
        ],
        out_specs=pl.BlockSpec((_R, d), lambda i: (i, 0)),
        out_shape=jax.ShapeDtypeStruct((N, d), jnp.float32),
    )(aggp, xw2, degp, b2)


def kernel(x, edge_index, edge_attr, W1, b1, W2, b2):
    row = edge_index[0].reshape(NUM_TILES, NCH, K)
    col = edge_index[1].reshape(NUM_TILES, NCH, K)
    ew = edge_attr.reshape(NUM_TILES, NCH, K)
    b1r = b1.reshape(1, D_HID)
    b2r = b2.reshape(1, D_OUT)

    degp = _deg_kernel(col, ew)
    xw1 = _matmul(x, W1)
    y1a, y1b = _prescale(xw1, degp)
    aggp1a = _msg_kernel_64(y1a, row, col, ew)
    aggp1b = _msg_kernel_64(y1b, row, col, ew)
    xw2, y2 = _layer(aggp1a, aggp1b, xw1, degp, b1r, W2)
    aggp2 = _msg_kernel_64(y2, row, col, ew)
    out = _final(aggp2, xw2, degp, b2r)
    return out


# speedup vs baseline: 13.0553x; 1.3907x over previous
import functools

import jax
import jax.numpy as jnp
from jax import lax
from jax.experimental import pallas as pl
from jax.experimental.pallas import tpu as pltpu
from jax.experimental.pallas import tpu_sc as plsc

N = 10000
E = 320000
D_IN = 128
D_HID = 128
D_OUT = 64

NUM_CORES = 2
NUM_SUBCORES = 16
NUM_TILES = NUM_CORES * NUM_SUBCORES
EPT = E // NUM_TILES
K = 80
NCH = EPT // K
RPT = N // NUM_SUBCORES
ZR = 125
WB = 624
WB_TAIL = N - NUM_SUBCORES * WB

_MESH = plsc.VectorSubcoreMesh(core_axis_name="c", subcore_axis_name="s")
_SC_PARAMS = pltpu.CompilerParams(use_tc_tiling_on_sc=False)


def _zero_acc(zbuf, acc, sid, d):
    def _zrow(i, carry):
        for j in range(d // 16):
            zbuf[i, pl.ds(j * 16, 16)] = jnp.zeros((16,), jnp.float32)
        return carry

    lax.fori_loop(0, ZR, _zrow, 0)
    for z in range(RPT // ZR):
        pltpu.sync_copy(zbuf, acc.at[pl.ds(sid * RPT + z * ZR, ZR)])


def _deg_body(col_hbm, ew_hbm, out_hbm, cidx, ewv, rows, zbuf, acc, sem):
    cid = lax.axis_index("c")
    sid = lax.axis_index("s")
    wid = cid * NUM_SUBCORES + sid

    _zero_acc(zbuf, acc, sid, 16)
    plsc.subcore_barrier()

    pltpu.sync_copy(col_hbm.at[wid], cidx)
    pltpu.sync_copy(ew_hbm.at[wid], ewv)

    def chunk(i, carry):
        def fill(g, c2):
            wv = ewv[i, pl.ds(g * 16, 16)]
            for j in range(16):
                rows[g * 16 + j, :] = jnp.full((16,), wv[j], jnp.float32)
            return c2

        lax.fori_loop(0, K // 16, fill, 0)
        pltpu.sync_copy(rows, acc.at[cidx.at[i]], add=True)
        return carry

    lax.fori_loop(0, NCH, chunk, 0)
    plsc.subcore_barrier()
    pltpu.sync_copy(acc.at[pl.ds(sid * WB, WB)],
                    out_hbm.at[cid, pl.ds(sid * WB, WB)])

    @pl.when(sid == 0)
    def _tail():
        pltpu.sync_copy(acc.at[pl.ds(NUM_SUBCORES * WB, WB_TAIL)],
                        out_hbm.at[cid, pl.ds(NUM_SUBCORES * WB, WB_TAIL)])


_deg_kernel = functools.partial(
    pl.kernel,
    out_type=jax.ShapeDtypeStruct((NUM_CORES, N, 16), jnp.float32),
    mesh=_MESH,
    scratch_types=[
        pltpu.VMEM((NCH, K), jnp.int32),
        pltpu.VMEM((NCH, K), jnp.float32),
        pltpu.VMEM((K, 16), jnp.float32),
        pltpu.VMEM((ZR, 16), jnp.float32),
        pltpu.VMEM_SHARED((N, 16), jnp.float32),
        pltpu.SemaphoreType.DMA,
    ],
    compiler_params=_SC_PARAMS,
)(_deg_body)


def _msg_body(d, y_hbm, row_hbm, col_hbm, ew_hbm, out_hbm,
              ridx, cidx, ewv, rows0, rows1, zbuf, acc, sem0, sem1):
    cid = lax.axis_index("c")
    sid = lax.axis_index("s")
    wid = cid * NUM_SUBCORES + sid

    _zero_acc(zbuf, acc, sid, d)
    plsc.subcore_barrier()

    pltpu.sync_copy(row_hbm.at[wid], ridx)
    pltpu.sync_copy(col_hbm.at[wid], cidx)
    pltpu.sync_copy(ew_hbm.at[wid], ewv)

    def start(i, buf, sem):
        pltpu.make_async_copy(y_hbm.at[ridx.at[i]], buf, sem).start()

    def drain(i, buf, sem):
        pltpu.make_async_copy(y_hbm.at[ridx.at[i]], buf, sem).wait()

    def process(i, buf):
        def scale(g, c2):
            wv = ewv[i, pl.ds(g * 16, 16)]
            for u in range(16):
                e = g * 16 + u
                w = wv[u]
                for j in range(d // 16):
                    buf[e, pl.ds(j * 16, 16)] = (
                        buf[e, pl.ds(j * 16, 16)] * w)
            return c2

        lax.fori_loop(0, K // 16, scale, 0)
        pltpu.sync_copy(buf, acc.at[cidx.at[i]], add=True)

    start(0, rows0, sem0)

    def pair(t, carry):
        i0 = 2 * t
        drain(i0, rows0, sem0)
        start(i0 + 1, rows1, sem1)
        process(i0, rows0)
        drain(i0 + 1, rows1, sem1)
        start(i0 + 2, rows0, sem0)
        process(i0 + 1, rows1)
        return carry

    lax.fori_loop(0, (NCH - 1) // 2, pair, 0)
    drain(NCH - 1, rows0, sem0)
    process(NCH - 1, rows0)
    plsc.subcore_barrier()
    pltpu.sync_copy(acc.at[pl.ds(sid * WB, WB)],
                    out_hbm.at[cid, pl.ds(sid * WB, WB)])

    @pl.when(sid == 0)
    def _tail():
        pltpu.sync_copy(acc.at[pl.ds(NUM_SUBCORES * WB, WB_TAIL)],
                        out_hbm.at[cid, pl.ds(NUM_SUBCORES * WB, WB_TAIL)])


def _make_msg_kernel(d):
    return functools.partial(
        pl.kernel,
        out_type=jax.ShapeDtypeStruct((NUM_CORES, N, d), jnp.float32),
        mesh=_MESH,
        scratch_types=[
            pltpu.VMEM((NCH, K), jnp.int32),
            pltpu.VMEM((NCH, K), jnp.int32),
            pltpu.VMEM((NCH, K), jnp.float32),
            pltpu.VMEM((K, d), jnp.float32),
            pltpu.VMEM((K, d), jnp.float32),
            pltpu.VMEM((ZR, d), jnp.float32),
            pltpu.VMEM_SHARED((N, d), jnp.float32),
            pltpu.SemaphoreType.DMA,
            pltpu.SemaphoreType.DMA,
        ],
        compiler_params=_SC_PARAMS,
    )(functools.partial(_msg_body, d))


_msg_kernel_64 = _make_msg_kernel(D_OUT)


_R = 1000
_G = N // _R


def _mm_body(x_ref, w_ref, o_ref):
    o_ref[...] = jnp.dot(x_ref[...], w_ref[...],
                         preferred_element_type=jnp.float32)


def _matmul(x, w):
    return pl.pallas_call(
        _mm_body,
        grid=(_G,),
        in_specs=[
            pl.BlockSpec((_R, x.shape[1]), lambda i: (i, 0)),
            pl.BlockSpec(w.shape, lambda i: (0, 0)),
        ],
        out_specs=pl.BlockSpec((_R, w.shape[1]), lambda i: (i, 0)),
        out_shape=jax.ShapeDtypeStruct((x.shape[0], w.shape[1]),
                                       jnp.float32),
    )(x, w)


def _dinv_of(degp_ref):
    deg = degp_ref[0, :, :1] + degp_ref[1, :, :1] + 1.0
    return lax.rsqrt(deg)


def _prescale_body(xw_ref, degp_ref, ya_ref, yb_ref):
    y = xw_ref[...] * _dinv_of(degp_ref)
    ya_ref[...] = y[:, :D_OUT]
    yb_ref[...] = y[:, D_OUT:]


def _prescale(xw, degp):
    d = xw.shape[1]
    return pl.pallas_call(
        _prescale_body,
        grid=(_G,),
        in_specs=[
            pl.BlockSpec((_R, d), lambda i: (i, 0)),
            pl.BlockSpec((NUM_CORES, _R, 16), lambda i: (0, i, 0)),
        ],
        out_specs=[
            pl.BlockSpec((_R, D_OUT), lambda i: (i, 0)),
            pl.BlockSpec((_R, D_OUT), lambda i: (i, 0)),
        ],
        out_shape=[
            jax.ShapeDtypeStruct((N, D_OUT), jnp.float32),
            jax.ShapeDtypeStruct((N, D_OUT), jnp.float32),
        ],
    )(xw, degp)


def _layer_body(aggpa_ref, aggpb_ref, xw_ref, degp_ref, b_ref, w_ref,
                xw2_ref, y2_ref):
    dinv = _dinv_of(degp_ref)
    agg = jnp.concatenate(
        [aggpa_ref[0] + aggpa_ref[1], aggpb_ref[0] + aggpb_ref[1]], axis=1)
    s = dinv * agg + (dinv * dinv) * xw_ref[...] + b_ref[...]
    h = jnp.maximum(s, 0.0)
    xw2 = jnp.dot(h, w_ref[...], preferred_element_type=jnp.float32)
    xw2_ref[...] = xw2
    y2_ref[...] = xw2 * dinv


def _layer(aggpa, aggpb, xw1, degp, b1, w2):
    d_in = xw1.shape[1]
    d_out = w2.shape[1]
    return pl.pallas_call(
        _layer_body,
        grid=(_G,),
        in_specs=[
            pl.BlockSpec((NUM_CORES, _R, D_OUT), lambda i: (0, i, 0)),
            pl.BlockSpec((NUM_CORES, _R, D_OUT), lambda i: (0, i, 0)),
            pl.BlockSpec((_R, d_in), lambda i: (i, 0)),
            pl.BlockSpec((NUM_CORES, _R, 16), lambda i: (0, i, 0)),
            pl.BlockSpec((1, d_in), lambda i: (0, 0)),
            pl.BlockSpec((d_in, d_out), lambda i: (0, 0)),
        ],
        out_specs=[
            pl.BlockSpec((_R, d_out), lambda i: (i, 0)),
            pl.BlockSpec((_R, d_out), lambda i: (i, 0)),
        ],
        out_shape=[
            jax.ShapeDtypeStruct((N, d_out), jnp.float32),
            jax.ShapeDtypeStruct((N, d_out), jnp.float32),
        ],
    )(aggpa, aggpb, xw1, degp, b1, w2)


def _final_body(aggp_ref, xw_ref, degp_ref, b_ref, o_ref):
    dinv = _dinv_of(degp_ref)
    o_ref[...] = (dinv * (aggp_ref[0] + aggp_ref[1])
                  + (dinv * dinv) * xw_ref[...] + b_ref[...])


def _final(aggp, xw2, degp, b2):
    d = xw2.shape[1]
    return pl.pallas_call(
        _final_body,
        grid=(_G,),
        in_specs=[
            pl.BlockSpec((NUM_CORES, _R, d), lambda i: (0, i, 0)),
            pl.BlockSpec((_R, d), lambda i: (i, 0)),
            pl.BlockSpec((NUM_CORES, _R, 16), lambda i: (0, i, 0)),
            pl.BlockSpec((1, d), lambda i: (0, 0)),
        ],
        out_specs=pl.BlockSpec((_R, d), lambda i: (i, 0)),
        out_shape=jax.ShapeDtypeStruct((N, d), jnp.float32),
    )(aggp, xw2, degp, b2)


def kernel(x, edge_index, edge_attr, W1, b1, W2, b2):
    row = edge_index[0].reshape(NUM_TILES, NCH, K)
    col = edge_index[1].reshape(NUM_TILES, NCH, K)
    ew = edge_attr.reshape(NUM_TILES, NCH, K)
    b1r = b1.reshape(1, D_HID)
    b2r = b2.reshape(1, D_OUT)

    degp = _deg_kernel(col, ew)
    xw1 = _matmul(x, W1)
    y1a, y1b = _prescale(xw1, degp)
    aggp1a = _msg_kernel_64(y1a, row, col, ew)
    aggp1b = _msg_kernel_64(y1b, row, col, ew)
    xw2, y2 = _layer(aggp1a, aggp1b, xw1, degp, b1r, W2)
    aggp2 = _msg_kernel_64(y2, row, col, ew)
    out = _final(aggp2, xw2, degp, b2r)
    return out
